# baseline (speedup 1.0000x reference)
import jax
import jax.numpy as jnp
from jax.experimental import pallas as pl
from jax.experimental.pallas import tpu as pltpu

B = 16384
INPUT_DIM = 768
EMBED_DIM = 64
HIDDEN_DIM = 512
CODEBOOK_SIZE = 1024
N_LAYERS = 3
LOSS_WEIGHT = 10.0
NGRP = CODEBOOK_SIZE // 8

BM = 512


def _prep_block(cb_ref, cbg_ref, cbnb_ref, hi_ref, mid_ref, lo_ref):
    cb = cb_ref[...]
    cb_n = cb / (jnp.sqrt(jnp.sum(cb * cb, axis=-1, keepdims=True)) + 1e-8)
    cbnb_ref[...] = cb_n.astype(jnp.bfloat16)

    cbg = cbg_ref[...]
    chunks = [cbg[:, :, j * EMBED_DIM:(j + 1) * EMBED_DIM] for j in range(8)]
    norms = [jnp.sqrt(jnp.sum(c * c, axis=-1, keepdims=True)) + 1e-8
             for c in chunks]
    cbg_n = jnp.concatenate([c / n for c, n in zip(chunks, norms)], axis=-1)
    hi = cbg_n.astype(jnp.bfloat16)
    r1 = cbg_n - hi.astype(jnp.float32)
    mid = r1.astype(jnp.bfloat16)
    lo = (r1 - mid.astype(jnp.float32)).astype(jnp.bfloat16)
    hi_ref[...] = hi
    mid_ref[...] = mid
    lo_ref[...] = lo


def _rqvae_block(x_ref, w1_ref, b1_ref, w2_ref, b2_ref, cbnb_ref,
                 hi_ref, mid_ref, lo_ref,
                 quant_ref, id0_ref, id1_ref, id2_ref, loss_ref):
    xb = x_ref[...].astype(jnp.bfloat16)
    h = jnp.maximum(jnp.dot(xb, w1_ref[...].astype(jnp.bfloat16),
                            preferred_element_type=jnp.float32) + b1_ref[...], 0.0)
    res = jnp.dot(h.astype(jnp.bfloat16), w2_ref[...].astype(jnp.bfloat16),
                  preferred_element_type=jnp.float32) + b2_ref[...]

    id_refs = (id0_ref, id1_ref, id2_ref)
    NH = 2
    HM = BM // NH
    residual = [res[h * HM:(h + 1) * HM] for h in range(NH)]
    quant_acc = [jnp.zeros((HM, EMBED_DIM), jnp.float32) for _ in range(NH)]
    loss_parts = []
    for q in range(N_LAYERS):
        r_n = [residual[h] / (jnp.sqrt(jnp.sum(residual[h] * residual[h],
                                               axis=-1, keepdims=True)) + 1e-8)
               for h in range(NH)]
        sim = [jax.lax.dot_general(r_n[h].astype(jnp.bfloat16), cbnb_ref[q],
                                   (((1,), (1,)), ((), ())),
                                   preferred_element_type=jnp.float32)
               for h in range(NH)]
        idx = [jnp.argmax(sim[h], axis=-1).astype(jnp.int32) for h in range(NH)]

        quant = []
        for h in range(NH):
            g = jax.lax.shift_right_logical(idx[h], 3)
            r = jnp.bitwise_and(idx[h], 7)[:, None]
            onehot_g = (g[:, None] ==
                        jax.lax.broadcasted_iota(jnp.int32, (HM, NGRP), 1)
                        ).astype(jnp.bfloat16)
            grp = (jnp.dot(onehot_g, hi_ref[q], preferred_element_type=jnp.float32)
                   + jnp.dot(onehot_g, mid_ref[q], preferred_element_type=jnp.float32)
                   + jnp.dot(onehot_g, lo_ref[q], preferred_element_type=jnp.float32))
            c = [grp[:, j * EMBED_DIM:(j + 1) * EMBED_DIM] for j in range(8)]
            b0 = jnp.bitwise_and(r, 1) != 0
            c = [jnp.where(b0, c[2 * j + 1], c[2 * j]) for j in range(4)]
            b1 = jnp.bitwise_and(r, 2) != 0
            c = [jnp.where(b1, c[2 * j + 1], c[2 * j]) for j in range(2)]
            b2 = jnp.bitwise_and(r, 4) != 0
            quant.append(jnp.where(b2, c[1], c[0]))

        for h in range(NH):
            diff = quant[h] - residual[h]
            loss_parts.append(jnp.sum(diff * diff))
            quant_acc[h] = quant_acc[h] + quant[h]
            residual[h] = residual[h] - quant[h]
        for h in range(NH):
            id_refs[q][h * HM:(h + 1) * HM] = idx[h]

    quant_ref[...] = jnp.concatenate(quant_acc, axis=0)
    lane = jax.lax.broadcasted_iota(jnp.int32, (1, 1, 128), 2)
    vec = jnp.zeros((1, 1, 128), jnp.float32)
    for q in range(N_LAYERS):
        part = loss_parts[NH * q]
        for h in range(1, NH):
            part = part + loss_parts[NH * q + h]
        vec = vec + jnp.where(lane == q, part, 0.0)
    loss_ref[...] = vec


@jax.jit
def kernel(x, enc_w1, enc_b1, enc_w2, enc_b2, codebooks):
    cbnb, cb_hi, cb_mid, cb_lo = pl.pallas_call(
        _prep_block,
        out_shape=(
            jax.ShapeDtypeStruct((N_LAYERS, CODEBOOK_SIZE, EMBED_DIM), jnp.bfloat16),
            jax.ShapeDtypeStruct((N_LAYERS, NGRP, 8 * EMBED_DIM), jnp.bfloat16),
            jax.ShapeDtypeStruct((N_LAYERS, NGRP, 8 * EMBED_DIM), jnp.bfloat16),
            jax.ShapeDtypeStruct((N_LAYERS, NGRP, 8 * EMBED_DIM), jnp.bfloat16),
        ),
    )(codebooks, codebooks.reshape(N_LAYERS, NGRP, 8 * EMBED_DIM))

    grid = B // BM
    out_shapes = (
        jax.ShapeDtypeStruct((B, EMBED_DIM), jnp.float32),
        jax.ShapeDtypeStruct((B,), jnp.int32),
        jax.ShapeDtypeStruct((B,), jnp.int32),
        jax.ShapeDtypeStruct((B,), jnp.int32),
        jax.ShapeDtypeStruct((grid, 1, 128), jnp.float32),
    )
    in_specs = [
        pl.BlockSpec((BM, INPUT_DIM), lambda i: (i, 0)),
        pl.BlockSpec((INPUT_DIM, HIDDEN_DIM), lambda i: (0, 0)),
        pl.BlockSpec((1, HIDDEN_DIM), lambda i: (0, 0)),
        pl.BlockSpec((HIDDEN_DIM, EMBED_DIM), lambda i: (0, 0)),
        pl.BlockSpec((1, EMBED_DIM), lambda i: (0, 0)),
        pl.BlockSpec((N_LAYERS, CODEBOOK_SIZE, EMBED_DIM), lambda i: (0, 0, 0)),
        pl.BlockSpec((N_LAYERS, NGRP, 8 * EMBED_DIM), lambda i: (0, 0, 0)),
        pl.BlockSpec((N_LAYERS, NGRP, 8 * EMBED_DIM), lambda i: (0, 0, 0)),
        pl.BlockSpec((N_LAYERS, NGRP, 8 * EMBED_DIM), lambda i: (0, 0, 0)),
    ]
    out_specs = (
        pl.BlockSpec((BM, EMBED_DIM), lambda i: (i, 0)),
        pl.BlockSpec((BM,), lambda i: (i,)),
        pl.BlockSpec((BM,), lambda i: (i,)),
        pl.BlockSpec((BM,), lambda i: (i,)),
        pl.BlockSpec((1, 1, 128), lambda i: (i, 0, 0)),
    )
    quant_out, i0, i1, i2, loss_parts = pl.pallas_call(
        _rqvae_block,
        grid=(grid,),
        in_specs=in_specs,
        out_specs=out_specs,
        out_shape=out_shapes,
        compiler_params=pltpu.CompilerParams(
            dimension_semantics=("parallel",)),
    )(x, enc_w1, enc_b1.reshape(1, HIDDEN_DIM), enc_w2,
      enc_b2.reshape(1, EMBED_DIM), cbnb, cb_hi, cb_mid, cb_lo)

    sem_ids = jnp.stack([i0, i1, i2], axis=-1)
    loss = (LOSS_WEIGHT / (B * EMBED_DIM)) * jnp.sum(loss_parts[:, 0, :N_LAYERS])
    return (loss, sem_ids, quant_out)


# speedup vs baseline: 1.3630x; 1.3630x over previous
import jax
import jax.numpy as jnp
from jax.experimental import pallas as pl
from jax.experimental.pallas import tpu as pltpu

B = 16384
INPUT_DIM = 768
EMBED_DIM = 64
HIDDEN_DIM = 512
CODEBOOK_SIZE = 1024
N_LAYERS = 3
LOSS_WEIGHT = 10.0
NGRP = CODEBOOK_SIZE // 8

BM = 512


def _prep_block(cb_ref, cbg_ref, cbnb_ref, hi_ref, mid_ref, lo_ref):
    cb = cb_ref[...]
    cb_n = cb / (jnp.sqrt(jnp.sum(cb * cb, axis=-1, keepdims=True)) + 1e-8)
    cbnb_ref[...] = cb_n.astype(jnp.bfloat16)

    cbg = cbg_ref[...]
    chunks = [cbg[:, :, j * EMBED_DIM:(j + 1) * EMBED_DIM] for j in range(8)]
    norms = [jnp.sqrt(jnp.sum(c * c, axis=-1, keepdims=True)) + 1e-8
             for c in chunks]
    cbg_n = jnp.concatenate([c / n for c, n in zip(chunks, norms)], axis=-1)
    hi = cbg_n.astype(jnp.bfloat16)
    r1 = cbg_n - hi.astype(jnp.float32)
    mid = r1.astype(jnp.bfloat16)
    lo = (r1 - mid.astype(jnp.float32)).astype(jnp.bfloat16)
    hi_ref[...] = hi
    mid_ref[...] = mid
    lo_ref[...] = lo


def _rqvae_block(x_ref, w1_ref, b1_ref, w2_ref, b2_ref, cbnb_ref,
                 hi_ref, mid_ref, lo_ref,
                 quant_ref, id0_ref, id1_ref, id2_ref, loss_ref):
    xb = x_ref[...].astype(jnp.bfloat16)
    h = jnp.maximum(jnp.dot(xb, w1_ref[...].astype(jnp.bfloat16),
                            preferred_element_type=jnp.float32) + b1_ref[...], 0.0)
    res = jnp.dot(h.astype(jnp.bfloat16), w2_ref[...].astype(jnp.bfloat16),
                  preferred_element_type=jnp.float32) + b2_ref[...]

    id_refs = (id0_ref, id1_ref, id2_ref)
    NH = 2
    HM = BM // NH
    residual = [res[h * HM:(h + 1) * HM] for h in range(NH)]
    quant_acc = [jnp.zeros((HM, EMBED_DIM), jnp.float32) for _ in range(NH)]
    loss_parts = []
    for q in range(N_LAYERS):
        r_n = [residual[h] / (jnp.sqrt(jnp.sum(residual[h] * residual[h],
                                               axis=-1, keepdims=True)) + 1e-8)
               for h in range(NH)]
        sim = [jax.lax.dot_general(r_n[h].astype(jnp.bfloat16), cbnb_ref[q],
                                   (((1,), (1,)), ((), ())),
                                   preferred_element_type=jnp.float32)
               for h in range(NH)]
        idx = [jnp.argmax(sim[h], axis=-1).astype(jnp.int32) for h in range(NH)]

        quant = []
        for h in range(NH):
            g = jax.lax.shift_right_logical(idx[h], 3)
            r = jnp.bitwise_and(idx[h], 7)
            onehot_g = (g[:, None] ==
                        jax.lax.broadcasted_iota(jnp.int32, (HM, NGRP), 1)
                        ).astype(jnp.bfloat16)
            grp = (jnp.dot(onehot_g, hi_ref[q], preferred_element_type=jnp.float32)
                   + jnp.dot(onehot_g, mid_ref[q], preferred_element_type=jnp.float32)
                   + jnp.dot(onehot_g, lo_ref[q], preferred_element_type=jnp.float32))
            qh = jnp.zeros((HM, EMBED_DIM), jnp.float32)
            for j in range(8):
                qh = qh + jnp.where(
                    r[:, None] == j, grp[:, j * EMBED_DIM:(j + 1) * EMBED_DIM], 0.0)
            quant.append(qh)

        for h in range(NH):
            diff = quant[h] - residual[h]
            loss_parts.append(jnp.sum(diff * diff))
            quant_acc[h] = quant_acc[h] + quant[h]
            residual[h] = residual[h] - quant[h]
        for h in range(NH):
            id_refs[q][h * HM:(h + 1) * HM] = idx[h]

    quant_ref[...] = jnp.concatenate(quant_acc, axis=0)
    lane = jax.lax.broadcasted_iota(jnp.int32, (1, 1, 128), 2)
    vec = jnp.zeros((1, 1, 128), jnp.float32)
    for q in range(N_LAYERS):
        part = loss_parts[NH * q]
        for h in range(1, NH):
            part = part + loss_parts[NH * q + h]
        vec = vec + jnp.where(lane == q, part, 0.0)
    loss_ref[...] = vec


@jax.jit
def kernel(x, enc_w1, enc_b1, enc_w2, enc_b2, codebooks):
    cbnb, cb_hi, cb_mid, cb_lo = pl.pallas_call(
        _prep_block,
        out_shape=(
            jax.ShapeDtypeStruct((N_LAYERS, CODEBOOK_SIZE, EMBED_DIM), jnp.bfloat16),
            jax.ShapeDtypeStruct((N_LAYERS, NGRP, 8 * EMBED_DIM), jnp.bfloat16),
            jax.ShapeDtypeStruct((N_LAYERS, NGRP, 8 * EMBED_DIM), jnp.bfloat16),
            jax.ShapeDtypeStruct((N_LAYERS, NGRP, 8 * EMBED_DIM), jnp.bfloat16),
        ),
    )(codebooks, codebooks.reshape(N_LAYERS, NGRP, 8 * EMBED_DIM))

    grid = B // BM
    out_shapes = (
        jax.ShapeDtypeStruct((B, EMBED_DIM), jnp.float32),
        jax.ShapeDtypeStruct((B,), jnp.int32),
        jax.ShapeDtypeStruct((B,), jnp.int32),
        jax.ShapeDtypeStruct((B,), jnp.int32),
        jax.ShapeDtypeStruct((grid, 1, 128), jnp.float32),
    )
    in_specs = [
        pl.BlockSpec((BM, INPUT_DIM), lambda i: (i, 0)),
        pl.BlockSpec((INPUT_DIM, HIDDEN_DIM), lambda i: (0, 0)),
        pl.BlockSpec((1, HIDDEN_DIM), lambda i: (0, 0)),
        pl.BlockSpec((HIDDEN_DIM, EMBED_DIM), lambda i: (0, 0)),
        pl.BlockSpec((1, EMBED_DIM), lambda i: (0, 0)),
        pl.BlockSpec((N_LAYERS, CODEBOOK_SIZE, EMBED_DIM), lambda i: (0, 0, 0)),
        pl.BlockSpec((N_LAYERS, NGRP, 8 * EMBED_DIM), lambda i: (0, 0, 0)),
        pl.BlockSpec((N_LAYERS, NGRP, 8 * EMBED_DIM), lambda i: (0, 0, 0)),
        pl.BlockSpec((N_LAYERS, NGRP, 8 * EMBED_DIM), lambda i: (0, 0, 0)),
    ]
    out_specs = (
        pl.BlockSpec((BM, EMBED_DIM), lambda i: (i, 0)),
        pl.BlockSpec((BM,), lambda i: (i,)),
        pl.BlockSpec((BM,), lambda i: (i,)),
        pl.BlockSpec((BM,), lambda i: (i,)),
        pl.BlockSpec((1, 1, 128), lambda i: (i, 0, 0)),
    )
    quant_out, i0, i1, i2, loss_parts = pl.pallas_call(
        _rqvae_block,
        grid=(grid,),
        in_specs=in_specs,
        out_specs=out_specs,
        out_shape=out_shapes,
        compiler_params=pltpu.CompilerParams(
            dimension_semantics=("parallel",)),
    )(x, enc_w1, enc_b1.reshape(1, HIDDEN_DIM), enc_w2,
      enc_b2.reshape(1, EMBED_DIM), cbnb, cb_hi, cb_mid, cb_lo)

    sem_ids = jnp.stack([i0, i1, i2], axis=-1)
    loss = (LOSS_WEIGHT / (B * EMBED_DIM)) * jnp.sum(loss_parts[:, 0, :N_LAYERS])
    return (loss, sem_ids, quant_out)


# speedup vs baseline: 1.4051x; 1.0309x over previous
import jax
import jax.numpy as jnp
from jax.experimental import pallas as pl
from jax.experimental.pallas import tpu as pltpu

B = 16384
INPUT_DIM = 768
EMBED_DIM = 64
HIDDEN_DIM = 512
CODEBOOK_SIZE = 1024
N_LAYERS = 3
LOSS_WEIGHT = 10.0
NGRP = CODEBOOK_SIZE // 8

BM = 1024


def _prep_block(cb_ref, cbg_ref, cbnb_ref, hi_ref, mid_ref, lo_ref):
    cb = cb_ref[...]
    cb_n = cb / (jnp.sqrt(jnp.sum(cb * cb, axis=-1, keepdims=True)) + 1e-8)
    cbnb_ref[...] = cb_n.astype(jnp.bfloat16)

    cbg = cbg_ref[...]
    chunks = [cbg[:, :, j * EMBED_DIM:(j + 1) * EMBED_DIM] for j in range(8)]
    norms = [jnp.sqrt(jnp.sum(c * c, axis=-1, keepdims=True)) + 1e-8
             for c in chunks]
    cbg_n = jnp.concatenate([c / n for c, n in zip(chunks, norms)], axis=-1)
    hi = cbg_n.astype(jnp.bfloat16)
    r1 = cbg_n - hi.astype(jnp.float32)
    mid = r1.astype(jnp.bfloat16)
    lo = (r1 - mid.astype(jnp.float32)).astype(jnp.bfloat16)
    hi_ref[...] = hi
    mid_ref[...] = mid
    lo_ref[...] = lo


def _rqvae_block(x_ref, w1_ref, b1_ref, w2_ref, b2_ref, cbnb_ref,
                 hi_ref, mid_ref, lo_ref,
                 quant_ref, id0_ref, id1_ref, id2_ref, loss_ref):
    xb = x_ref[...].astype(jnp.bfloat16)
    h = jnp.maximum(jnp.dot(xb, w1_ref[...].astype(jnp.bfloat16),
                            preferred_element_type=jnp.float32) + b1_ref[...], 0.0)
    res = jnp.dot(h.astype(jnp.bfloat16), w2_ref[...].astype(jnp.bfloat16),
                  preferred_element_type=jnp.float32) + b2_ref[...]

    id_refs = (id0_ref, id1_ref, id2_ref)
    NH = 2
    HM = BM // NH
    residual = [res[h * HM:(h + 1) * HM] for h in range(NH)]
    quant_acc = [jnp.zeros((HM, EMBED_DIM), jnp.float32) for _ in range(NH)]
    loss_parts = []
    for q in range(N_LAYERS):
        r_n = [residual[h] / (jnp.sqrt(jnp.sum(residual[h] * residual[h],
                                               axis=-1, keepdims=True)) + 1e-8)
               for h in range(NH)]
        sim = [jax.lax.dot_general(r_n[h].astype(jnp.bfloat16), cbnb_ref[q],
                                   (((1,), (1,)), ((), ())),
                                   preferred_element_type=jnp.float32)
               for h in range(NH)]
        idx = [jnp.argmax(sim[h], axis=-1).astype(jnp.int32) for h in range(NH)]

        quant = []
        for h in range(NH):
            g = jax.lax.shift_right_logical(idx[h], 3)
            r = jnp.bitwise_and(idx[h], 7)
            onehot_g = (g[:, None] ==
                        jax.lax.broadcasted_iota(jnp.int32, (HM, NGRP), 1)
                        ).astype(jnp.bfloat16)
            grp = (jnp.dot(onehot_g, hi_ref[q], preferred_element_type=jnp.float32)
                   + jnp.dot(onehot_g, mid_ref[q], preferred_element_type=jnp.float32)
                   + jnp.dot(onehot_g, lo_ref[q], preferred_element_type=jnp.float32))
            qh = jnp.zeros((HM, EMBED_DIM), jnp.float32)
            for j in range(8):
                qh = qh + jnp.where(
                    r[:, None] == j, grp[:, j * EMBED_DIM:(j + 1) * EMBED_DIM], 0.0)
            quant.append(qh)

        for h in range(NH):
            diff = quant[h] - residual[h]
            loss_parts.append(jnp.sum(diff * diff))
            quant_acc[h] = quant_acc[h] + quant[h]
            residual[h] = residual[h] - quant[h]
        for h in range(NH):
            id_refs[q][h * HM:(h + 1) * HM] = idx[h]

    quant_ref[...] = jnp.concatenate(quant_acc, axis=0)
    lane = jax.lax.broadcasted_iota(jnp.int32, (1, 1, 128), 2)
    vec = jnp.zeros((1, 1, 128), jnp.float32)
    for q in range(N_LAYERS):
        part = loss_parts[NH * q]
        for h in range(1, NH):
            part = part + loss_parts[NH * q + h]
        vec = vec + jnp.where(lane == q, part, 0.0)
    loss_ref[...] = vec


@jax.jit
def kernel(x, enc_w1, enc_b1, enc_w2, enc_b2, codebooks):
    cbnb, cb_hi, cb_mid, cb_lo = pl.pallas_call(
        _prep_block,
        out_shape=(
            jax.ShapeDtypeStruct((N_LAYERS, CODEBOOK_SIZE, EMBED_DIM), jnp.bfloat16),
            jax.ShapeDtypeStruct((N_LAYERS, NGRP, 8 * EMBED_DIM), jnp.bfloat16),
            jax.ShapeDtypeStruct((N_LAYERS, NGRP, 8 * EMBED_DIM), jnp.bfloat16),
            jax.ShapeDtypeStruct((N_LAYERS, NGRP, 8 * EMBED_DIM), jnp.bfloat16),
        ),
    )(codebooks, codebooks.reshape(N_LAYERS, NGRP, 8 * EMBED_DIM))

    grid = B // BM
    out_shapes = (
        jax.ShapeDtypeStruct((B, EMBED_DIM), jnp.float32),
        jax.ShapeDtypeStruct((B,), jnp.int32),
        jax.ShapeDtypeStruct((B,), jnp.int32),
        jax.ShapeDtypeStruct((B,), jnp.int32),
        jax.ShapeDtypeStruct((grid, 1, 128), jnp.float32),
    )
    in_specs = [
        pl.BlockSpec((BM, INPUT_DIM), lambda i: (i, 0)),
        pl.BlockSpec((INPUT_DIM, HIDDEN_DIM), lambda i: (0, 0)),
        pl.BlockSpec((1, HIDDEN_DIM), lambda i: (0, 0)),
        pl.BlockSpec((HIDDEN_DIM, EMBED_DIM), lambda i: (0, 0)),
        pl.BlockSpec((1, EMBED_DIM), lambda i: (0, 0)),
        pl.BlockSpec((N_LAYERS, CODEBOOK_SIZE, EMBED_DIM), lambda i: (0, 0, 0)),
        pl.BlockSpec((N_LAYERS, NGRP, 8 * EMBED_DIM), lambda i: (0, 0, 0)),
        pl.BlockSpec((N_LAYERS, NGRP, 8 * EMBED_DIM), lambda i: (0, 0, 0)),
        pl.BlockSpec((N_LAYERS, NGRP, 8 * EMBED_DIM), lambda i: (0, 0, 0)),
    ]
    out_specs = (
        pl.BlockSpec((BM, EMBED_DIM), lambda i: (i, 0)),
        pl.BlockSpec((BM,), lambda i: (i,)),
        pl.BlockSpec((BM,), lambda i: (i,)),
        pl.BlockSpec((BM,), lambda i: (i,)),
        pl.BlockSpec((1, 1, 128), lambda i: (i, 0, 0)),
    )
    quant_out, i0, i1, i2, loss_parts = pl.pallas_call(
        _rqvae_block,
        grid=(grid,),
        in_specs=in_specs,
        out_specs=out_specs,
        out_shape=out_shapes,
        compiler_params=pltpu.CompilerParams(
            dimension_semantics=("parallel",)),
    )(x, enc_w1, enc_b1.reshape(1, HIDDEN_DIM), enc_w2,
      enc_b2.reshape(1, EMBED_DIM), cbnb, cb_hi, cb_mid, cb_lo)

    sem_ids = jnp.stack([i0, i1, i2], axis=-1)
    loss = (LOSS_WEIGHT / (B * EMBED_DIM)) * jnp.sum(loss_parts[:, 0, :N_LAYERS])
    return (loss, sem_ids, quant_out)
